# baseline (speedup 1.0000x reference)
import functools
import math

import jax
import jax.numpy as jnp
from jax import lax
from jax.experimental import pallas as pl
from jax.experimental.pallas import tpu as pltpu
from jax.experimental.pallas import tpu_sc as plsc

VOCAB = 50257
D_MODEL = 768
BATCH = 4
SEQ = 2048

NC = 2
NS = 16
LANES = 16
NW = NC * NS
POS_PER_W = SEQ // NW
SP = 8
NSC = POS_PER_W // SP
QROWS = BATCH * SP
NBUF = 4
NPBUF = 3
LOOK = 2
VECS_PER_ROW = D_MODEL // LANES
SCALE = math.sqrt(D_MODEL)

_mesh = plsc.VectorSubcoreMesh(core_axis_name="c", subcore_axis_name="s")


@functools.partial(
    pl.kernel,
    out_type=jax.ShapeDtypeStruct((BATCH, SEQ, D_MODEL), jnp.float32),
    mesh=_mesh,
    scratch_types=[
        pltpu.VMEM((NSC, QROWS), jnp.int32),
        pltpu.VMEM((SP, D_MODEL), jnp.float32),
        pltpu.VMEM((SP, D_MODEL), jnp.float32),
        pltpu.VMEM((SP, D_MODEL), jnp.float32),
        pltpu.VMEM((QROWS, D_MODEL), jnp.float32),
        pltpu.VMEM((QROWS, D_MODEL), jnp.float32),
        pltpu.VMEM((QROWS, D_MODEL), jnp.float32),
        pltpu.VMEM((QROWS, D_MODEL), jnp.float32),
        pltpu.SemaphoreType.DMA,
        pltpu.SemaphoreType.DMA,
        pltpu.SemaphoreType.DMA,
        pltpu.SemaphoreType.DMA,
        pltpu.SemaphoreType.DMA,
        pltpu.SemaphoreType.DMA,
        pltpu.SemaphoreType.DMA,
        pltpu.SemaphoreType.DMA,
        pltpu.SemaphoreType.DMA,
        pltpu.SemaphoreType.DMA,
        pltpu.SemaphoreType.DMA,
    ],
)
def _emb_kernel(ids_hbm, tok_hbm, pos_hbm, out_hbm,
                idx_v, pv0, pv1, pv2, q0, q1, q2, q3,
                ps0, ps1, ps2, gs0, gs1, gs2, gs3, ws0, ws1, ws2, ws3):
    wid = lax.axis_index("s") * NC + lax.axis_index("c")
    s_base = wid * POS_PER_W
    poss = [pv0, pv1, pv2]
    psems = [ps0, ps1, ps2]
    quads = [q0, q1, q2, q3]
    gsems = [gs0, gs1, gs2, gs3]
    wsems = [ws0, ws1, ws2, ws3]

    pltpu.sync_copy(ids_hbm.at[wid], idx_v)

    def issue_gather(sc):
        bu = sc % NBUF
        return [pltpu.async_copy(tok_hbm.at[idx_v.at[sc]], quads[bu],
                                 gsems[bu])]

    def issue_pos(sc):
        pb = sc % NPBUF
        return pltpu.async_copy(
            pos_hbm.at[pl.ds(s_base + sc * SP, SP)], poss[pb], psems[pb])

    gathers = [None] * NSC
    pos_cps = [None] * NSC
    writes = [[None] * BATCH for _ in range(NSC)]
    for sc in range(LOOK):
        gathers[sc] = issue_gather(sc)
        pos_cps[sc] = issue_pos(sc)

    for sc in range(NSC):
        bu = sc % NBUF
        pb = sc % NPBUF
        nxt = sc + LOOK
        if nxt < NSC:
            if nxt >= NBUF:
                for wcp in writes[nxt - NBUF]:
                    wcp.wait()
            gathers[nxt] = issue_gather(nxt)
            pos_cps[nxt] = issue_pos(nxt)
        for gcp in gathers[sc]:
            gcp.wait()
        pos_cps[sc].wait()

        def vec_body(i, bu=bu, pb=pb):
            r = i & (SP - 1)
            l = i >> 3
            sl = pl.ds(l * LANES, LANES)
            pv = poss[pb][r, sl]
            q = quads[bu]
            for b in range(BATCH):
                q[b * SP + r, sl] = q[b * SP + r, sl] * SCALE + pv

        plsc.parallel_loop(0, SP * VECS_PER_ROW, unroll=4)(vec_body)

        for b in range(BATCH):
            writes[sc][b] = pltpu.async_copy(
                quads[bu].at[pl.ds(b * SP, SP)],
                out_hbm.at[b, pl.ds(s_base + sc * SP, SP)],
                wsems[bu])

    for sc in range(NSC - NBUF, NSC):
        for wcp in writes[sc]:
            wcp.wait()


def kernel(token_ids, tok_table, pos_table):
    ids = jnp.reshape(token_ids.astype(jnp.int32), (BATCH, NW, NSC, SP))
    ids = jnp.transpose(ids, (1, 2, 0, 3)).reshape(NW, NSC, QROWS)
    return _emb_kernel(ids, tok_table, pos_table)


# speedup vs baseline: 1.0156x; 1.0156x over previous
import functools
import math

import jax
import jax.numpy as jnp
from jax import lax
from jax.experimental import pallas as pl
from jax.experimental.pallas import tpu as pltpu
from jax.experimental.pallas import tpu_sc as plsc

VOCAB = 50257
D_MODEL = 768
BATCH = 4
SEQ = 2048

NC = 2
NS = 16
LANES = 16
NW = NC * NS
POS_PER_W = SEQ // NW
SP = 8
NSC = POS_PER_W // SP
QROWS = BATCH * SP
NBUF = 4
NPBUF = 3
LOOK = 2
VECS_PER_ROW = D_MODEL // LANES
SCALE = math.sqrt(D_MODEL)

_mesh = plsc.VectorSubcoreMesh(core_axis_name="c", subcore_axis_name="s")


@functools.partial(
    pl.kernel,
    out_type=jax.ShapeDtypeStruct((BATCH, SEQ, D_MODEL), jnp.float32),
    mesh=_mesh,
    scratch_types=[
        pltpu.VMEM((NSC, QROWS), jnp.int32),
        pltpu.VMEM((SP, D_MODEL), jnp.float32),
        pltpu.VMEM((SP, D_MODEL), jnp.float32),
        pltpu.VMEM((SP, D_MODEL), jnp.float32),
        pltpu.VMEM((QROWS, D_MODEL), jnp.float32),
        pltpu.VMEM((QROWS, D_MODEL), jnp.float32),
        pltpu.VMEM((QROWS, D_MODEL), jnp.float32),
        pltpu.VMEM((QROWS, D_MODEL), jnp.float32),
        pltpu.SemaphoreType.DMA,
        pltpu.SemaphoreType.DMA,
        pltpu.SemaphoreType.DMA,
        pltpu.SemaphoreType.DMA,
        pltpu.SemaphoreType.DMA,
        pltpu.SemaphoreType.DMA,
        pltpu.SemaphoreType.DMA,
        pltpu.SemaphoreType.DMA,
        pltpu.SemaphoreType.DMA,
        pltpu.SemaphoreType.DMA,
        pltpu.SemaphoreType.DMA,
    ],
)
def _emb_kernel(ids_hbm, tok_hbm, pos_hbm, out_hbm,
                idx_v, pv0, pv1, pv2, q0, q1, q2, q3,
                ps0, ps1, ps2, gs0, gs1, gs2, gs3, ws0, ws1, ws2, ws3):
    wid = lax.axis_index("s") * NC + lax.axis_index("c")
    s_base = wid * POS_PER_W
    poss = [pv0, pv1, pv2]
    psems = [ps0, ps1, ps2]
    quads = [q0, q1, q2, q3]
    gsems = [gs0, gs1, gs2, gs3]
    wsems = [ws0, ws1, ws2, ws3]

    pltpu.sync_copy(ids_hbm.at[wid], idx_v)

    def issue_gather(sc):
        bu = sc % NBUF
        return [pltpu.async_copy(tok_hbm.at[idx_v.at[sc]], quads[bu],
                                 gsems[bu])]

    def issue_pos(sc):
        pb = sc % NPBUF
        return pltpu.async_copy(
            pos_hbm.at[pl.ds(s_base + sc * SP, SP)], poss[pb], psems[pb])

    gathers = [None] * NSC
    pos_cps = [None] * NSC
    writes = [[None] * BATCH for _ in range(NSC)]
    for sc in range(LOOK):
        gathers[sc] = issue_gather(sc)
        pos_cps[sc] = issue_pos(sc)

    for sc in range(NSC):
        bu = sc % NBUF
        pb = sc % NPBUF
        nxt = sc + LOOK
        if nxt < NSC:
            if nxt >= NBUF:
                for wcp in writes[nxt - NBUF]:
                    wcp.wait()
            gathers[nxt] = issue_gather(nxt)
            pos_cps[nxt] = issue_pos(nxt)
        for gcp in gathers[sc]:
            gcp.wait()
        pos_cps[sc].wait()

        def vec_body(i, bu=bu, pb=pb):
            r = i & (SP - 1)
            l = i >> 3
            sl = pl.ds(l * LANES, LANES)
            pv = poss[pb][r, sl]
            q = quads[bu]
            for b in range(BATCH):
                q[b * SP + r, sl] = q[b * SP + r, sl] * SCALE + pv

        plsc.parallel_loop(0, SP * VECS_PER_ROW, unroll=2)(vec_body)

        for b in range(BATCH):
            writes[sc][b] = pltpu.async_copy(
                quads[bu].at[pl.ds(b * SP, SP)],
                out_hbm.at[b, pl.ds(s_base + sc * SP, SP)],
                wsems[bu])

    for sc in range(NSC - NBUF, NSC):
        for wcp in writes[sc]:
            wcp.wait()


def kernel(token_ids, tok_table, pos_table):
    ids = jnp.reshape(token_ids.astype(jnp.int32), (BATCH, NW, NSC, SP))
    ids = jnp.transpose(ids, (1, 2, 0, 3)).reshape(NW, NSC, QROWS)
    return _emb_kernel(ids, tok_table, pos_table)


# speedup vs baseline: 1.0433x; 1.0273x over previous
import functools
import math

import jax
import jax.numpy as jnp
from jax import lax
from jax.experimental import pallas as pl
from jax.experimental.pallas import tpu as pltpu
from jax.experimental.pallas import tpu_sc as plsc

VOCAB = 50257
D_MODEL = 768
BATCH = 4
SEQ = 2048

NC = 2
NS = 16
LANES = 16
NW = NC * NS
POS_PER_W = SEQ // NW
SP = 8
NSC = POS_PER_W // SP
QROWS = BATCH * SP
NBUF = 4
NPBUF = 3
LOOK = 3
VECS_PER_ROW = D_MODEL // LANES
SCALE = math.sqrt(D_MODEL)

_mesh = plsc.VectorSubcoreMesh(core_axis_name="c", subcore_axis_name="s")


@functools.partial(
    pl.kernel,
    out_type=jax.ShapeDtypeStruct((BATCH, SEQ, D_MODEL), jnp.float32),
    mesh=_mesh,
    scratch_types=[
        pltpu.VMEM((NSC, QROWS), jnp.int32),
        pltpu.VMEM((SP, D_MODEL), jnp.float32),
        pltpu.VMEM((SP, D_MODEL), jnp.float32),
        pltpu.VMEM((SP, D_MODEL), jnp.float32),
        pltpu.VMEM((QROWS, D_MODEL), jnp.float32),
        pltpu.VMEM((QROWS, D_MODEL), jnp.float32),
        pltpu.VMEM((QROWS, D_MODEL), jnp.float32),
        pltpu.VMEM((QROWS, D_MODEL), jnp.float32),
        pltpu.SemaphoreType.DMA,
        pltpu.SemaphoreType.DMA,
        pltpu.SemaphoreType.DMA,
        pltpu.SemaphoreType.DMA,
        pltpu.SemaphoreType.DMA,
        pltpu.SemaphoreType.DMA,
        pltpu.SemaphoreType.DMA,
        pltpu.SemaphoreType.DMA,
        pltpu.SemaphoreType.DMA,
        pltpu.SemaphoreType.DMA,
        pltpu.SemaphoreType.DMA,
    ],
)
def _emb_kernel(ids_hbm, tok_hbm, pos_hbm, out_hbm,
                idx_v, pv0, pv1, pv2, q0, q1, q2, q3,
                ps0, ps1, ps2, gs0, gs1, gs2, gs3, ws0, ws1, ws2, ws3):
    wid = lax.axis_index("s") * NC + lax.axis_index("c")
    s_base = wid * POS_PER_W
    poss = [pv0, pv1, pv2]
    psems = [ps0, ps1, ps2]
    quads = [q0, q1, q2, q3]
    gsems = [gs0, gs1, gs2, gs3]
    wsems = [ws0, ws1, ws2, ws3]

    pltpu.sync_copy(ids_hbm.at[wid], idx_v)

    def issue_gather(sc):
        bu = sc % NBUF
        return [pltpu.async_copy(tok_hbm.at[idx_v.at[sc]], quads[bu],
                                 gsems[bu])]

    def issue_pos(sc):
        pb = sc % NPBUF
        return pltpu.async_copy(
            pos_hbm.at[pl.ds(s_base + sc * SP, SP)], poss[pb], psems[pb])

    gathers = [None] * NSC
    pos_cps = [None] * NSC
    writes = [[None] * BATCH for _ in range(NSC)]
    for sc in range(LOOK):
        gathers[sc] = issue_gather(sc)
        pos_cps[sc] = issue_pos(sc)

    for sc in range(NSC):
        bu = sc % NBUF
        pb = sc % NPBUF
        nxt = sc + LOOK
        if nxt < NSC:
            if nxt >= NBUF:
                for wcp in writes[nxt - NBUF]:
                    wcp.wait()
            gathers[nxt] = issue_gather(nxt)
            pos_cps[nxt] = issue_pos(nxt)
        for gcp in gathers[sc]:
            gcp.wait()
        pos_cps[sc].wait()

        def vec_body(i, bu=bu, pb=pb):
            r = i & (SP - 1)
            l = i >> 3
            sl = pl.ds(l * LANES, LANES)
            pv = poss[pb][r, sl]
            q = quads[bu]
            for b in range(BATCH):
                q[b * SP + r, sl] = q[b * SP + r, sl] * SCALE + pv

        plsc.parallel_loop(0, SP * VECS_PER_ROW, unroll=2)(vec_body)

        for b in range(BATCH):
            writes[sc][b] = pltpu.async_copy(
                quads[bu].at[pl.ds(b * SP, SP)],
                out_hbm.at[b, pl.ds(s_base + sc * SP, SP)],
                wsems[bu])

    for sc in range(NSC - NBUF, NSC):
        for wcp in writes[sc]:
            wcp.wait()


def kernel(token_ids, tok_table, pos_table):
    ids = jnp.reshape(token_ids.astype(jnp.int32), (BATCH, NW, NSC, SP))
    ids = jnp.transpose(ids, (1, 2, 0, 3)).reshape(NW, NSC, QROWS)
    return _emb_kernel(ids, tok_table, pos_table)
